# baseline (speedup 1.0000x reference)
import functools

import jax
import jax.numpy as jnp
from jax import lax
from jax.experimental import pallas as pl
from jax.experimental.pallas import tpu as pltpu
from jax.experimental.pallas import tpu_sc as plsc

_D = 64
_B = 16384
_NROW = 1_000_000
_MPAD = 1_000_064
_NC = 2
_NS = 16
_NW = _NC * _NS
_L = 16
_W = 512
_HALF = 499712
_TAILC = 64
_MLOC = 500608
_DBIN = 500480
_ZB = 8192

_mesh = plsc.VectorSubcoreMesh(core_axis_name="c", subcore_axis_name="s")


@functools.partial(
    pl.kernel,
    mesh=_mesh,
    out_type=jax.ShapeDtypeStruct((_NW, _D, _L), jnp.float32),
    scratch_types=[
        pltpu.VMEM_SHARED((_MLOC,), jnp.float32),
        pltpu.VMEM((8, 128), jnp.int32),
        pltpu.VMEM((128,), jnp.float32),
        pltpu.VMEM((_ZB,), jnp.float32),
        pltpu.VMEM((2, _D, _W), jnp.float32),
        pltpu.VMEM((_W,), jnp.float32),
        pltpu.VMEM((_D, _L), jnp.float32),
        pltpu.SemaphoreType.DMA,
        pltpu.SemaphoreType.DMA,
    ],
    compiler_params=pltpu.CompilerParams(use_tc_tiling_on_sc=True, needs_layout_passes=False),
)
def _count_matvec(idx_hbm, table_hbm, tail_hbm, out_hbm,
                  m_s, idx_v, ones_v, z_v, tbuf, mbuf, part_v, zsem, sem):
    cid = lax.axis_index("c")
    sid = lax.axis_index("s")
    wid = sid * _NC + cid
    base = cid * _HALF
    crange = jnp.where(cid == 0, _HALF, _NROW - _HALF)

    def zfill(i, _):
        z_v[pl.ds(i * _L, _L)] = jnp.zeros((_L,), jnp.float32)
        return 0
    lax.fori_loop(0, _ZB // _L, zfill, 0)
    for j in range(8):
        ones_v[pl.ds(j * _L, _L)] = jnp.ones((_L,), jnp.float32)

    mseg = _MLOC // _NS
    nzc = mseg // _ZB
    zrem = mseg - nzc * _ZB
    for j in range(nzc):
        pltpu.make_async_copy(
            z_v, m_s.at[pl.ds(sid * mseg + j * _ZB, _ZB)], zsem).start()
    pltpu.make_async_copy(
        z_v.at[pl.ds(0, zrem)],
        m_s.at[pl.ds(sid * mseg + nzc * _ZB, zrem)], zsem).start()
    pltpu.sync_copy(idx_hbm.at[sid], idx_v)
    for k in range(8):
        for q in range(8):
            v = idx_v[k, pl.ds(q * _L, _L)]
            loc = v - base
            ok = (loc >= 0) & (loc < crange)
            idx_v[k, pl.ds(q * _L, _L)] = jnp.where(ok, loc, _DBIN)
    for j in range(nzc):
        pltpu.make_async_copy(
            z_v, m_s.at[pl.ds(sid * mseg + j * _ZB, _ZB)], zsem).wait()
    pltpu.make_async_copy(
        z_v.at[pl.ds(0, zrem)],
        m_s.at[pl.ds(sid * mseg + nzc * _ZB, zrem)], zsem).wait()
    plsc.subcore_barrier()

    for k in range(8):
        pltpu.sync_copy(ones_v, m_s.at[idx_v.at[k]], add=True)
    plsc.subcore_barrier()

    def pzero(i, _):
        part_v[i] = jnp.zeros((_L,), jnp.float32)
        return 0
    lax.fori_loop(0, _D, pzero, 0)

    def chunk_col(k):
        return base + (sid + k * _NS) * _W

    def fire(k):
        pltpu.make_async_copy(
            table_hbm.at[:, pl.ds(chunk_col(k), _W)], tbuf.at[jnp.mod(k, 2)],
            sem).start()

    def accumulate(p, loc0):
        pltpu.sync_copy(m_s.at[pl.ds(loc0, _W)], mbuf)
        for j in range(_W // _L):
            mv = mbuf[pl.ds(j * _L, _L)]
            cnt = plsc.all_reduce_population_count(mv != 0.0)

            @pl.when(cnt[0] > 0)
            def _(mv=mv, j=j):
                def dgroup(dg, _):
                    for u in range(4):
                        d = dg * 4 + u
                        plsc.addupdate(
                            part_v.at[d],
                            tbuf[p, d, pl.ds(j * _L, _L)] * mv)
                    return 0
                lax.fori_loop(0, _D // 4, dgroup, 0)

    kpw = 61
    fire(0)

    def scan_body(k, _):
        pltpu.make_async_copy(
            table_hbm.at[:, pl.ds(0, _W)], tbuf.at[jnp.mod(k, 2)], sem
        ).wait()

        @pl.when(k + 1 < kpw)
        def _():
            fire(k + 1)
        accumulate(jnp.mod(k, 2), chunk_col(k) - base)
        return 0
    lax.fori_loop(0, kpw, scan_body, 0)

    @pl.when((cid == 1) & (sid == 0))
    def _():
        col0 = _HALF + 976 * _W
        pltpu.make_async_copy(
            table_hbm.at[:, pl.ds(col0, _W)], tbuf.at[0], sem).start()
        pltpu.make_async_copy(
            table_hbm.at[:, pl.ds(col0, _W)], tbuf.at[0], sem).wait()
        accumulate(0, col0 - _HALF)

    @pl.when((cid == 1) & (sid == 1))
    def _():
        loc0 = 999936 - _HALF
        pltpu.make_async_copy(
            tail_hbm, tbuf.at[0].at[:, pl.ds(0, 128)], sem).start()
        pltpu.make_async_copy(
            tail_hbm, tbuf.at[0].at[:, pl.ds(0, 128)], sem).wait()
        pltpu.sync_copy(m_s.at[pl.ds(loc0, 128)], mbuf.at[pl.ds(0, 128)])
        for j in range(128 // _L):
            mv = mbuf[pl.ds(j * _L, _L)]
            cnt = plsc.all_reduce_population_count(mv != 0.0)

            @pl.when(cnt[0] > 0)
            def _(mv=mv, j=j):
                def dtail(dg, _):
                    for u in range(4):
                        d = dg * 4 + u
                        plsc.addupdate(
                            part_v.at[d],
                            tbuf[0, d, pl.ds(j * _L, _L)] * mv)
                    return 0
                lax.fori_loop(0, _D // 4, dtail, 0)

    pltpu.sync_copy(part_v, out_hbm.at[wid])


def _tail_body(p_ref, w_ref, b_ref, o_ref):
    tot = jnp.sum(p_ref[...], axis=(0, 2))
    h = 1.0 / (1.0 + jnp.exp(-(tot * (1.0 / _B))))
    logits = jnp.sum(w_ref[...] * h[None, :], axis=1, keepdims=True)
    logits = logits + b_ref[...]
    s = 1.0 / (1.0 + jnp.exp(-logits))
    row = lax.broadcasted_iota(jnp.int32, (8, 1), 0)
    e = jnp.where(row < 2, jnp.exp(s), 0.0)
    o_ref[...] = e / jnp.sum(e)


def kernel(X, emb, W, b):
    idx = X.astype(jnp.int32).reshape(_NS, 8, 128)
    tail = jnp.zeros((_D, 128), jnp.float32).at[:, :_TAILC].set(
        emb[999936:].T)
    partials = _count_matvec(idx, emb.T, tail)
    wp = jnp.zeros((8, _D), jnp.float32).at[:2].set(W)
    bp = jnp.zeros((8, 1), jnp.float32).at[:2, 0].set(b)
    out = pl.pallas_call(
        _tail_body,
        out_shape=jax.ShapeDtypeStruct((8, 1), jnp.float32),
    )(partials, wp, bp)
    return out[:2, 0]


# speedup vs baseline: 1.5956x; 1.5956x over previous
import functools

import jax
import jax.numpy as jnp
from jax import lax
from jax.experimental import pallas as pl
from jax.experimental.pallas import tpu as pltpu
from jax.experimental.pallas import tpu_sc as plsc

_D = 64
_B = 16384
_NROW = 1_000_000
_MPAD = 1_000_064
_NC = 2
_NS = 16
_NW = _NC * _NS
_L = 16
_W = 512
_HALF = 499712
_TAILC = 64
_MLOC = 500608
_DBIN = 500480
_ZB = 8192

_mesh = plsc.VectorSubcoreMesh(core_axis_name="c", subcore_axis_name="s")


@functools.partial(
    pl.kernel,
    mesh=_mesh,
    out_type=jax.ShapeDtypeStruct((_MPAD,), jnp.float32),
    scratch_types=[
        pltpu.VMEM_SHARED((_MLOC,), jnp.float32),
        pltpu.VMEM((8, 128), jnp.int32),
        pltpu.VMEM((128,), jnp.float32),
        pltpu.VMEM((_ZB,), jnp.float32),
        pltpu.VMEM((31272,), jnp.float32),
        pltpu.SemaphoreType.DMA,
    ],
    compiler_params=pltpu.CompilerParams(use_tc_tiling_on_sc=True),
)
def _build_m(idx_hbm, m_hbm, m_s, idx_v, ones_v, z_v, bounce_v, zsem):
    cid = lax.axis_index("c")
    sid = lax.axis_index("s")
    wid = sid * _NC + cid
    base = cid * _HALF
    crange = jnp.where(cid == 0, _HALF, _NROW - _HALF)

    def zfill(i, _):
        z_v[pl.ds(i * _L, _L)] = jnp.zeros((_L,), jnp.float32)
        return 0
    lax.fori_loop(0, _ZB // _L, zfill, 0)
    for j in range(8):
        ones_v[pl.ds(j * _L, _L)] = jnp.ones((_L,), jnp.float32)

    mseg = _MLOC // _NS
    nzc = mseg // _ZB
    zrem = mseg - nzc * _ZB
    for j in range(nzc):
        pltpu.make_async_copy(
            z_v, m_s.at[pl.ds(sid * mseg + j * _ZB, _ZB)], zsem).start()
    pltpu.make_async_copy(
        z_v.at[pl.ds(0, zrem)],
        m_s.at[pl.ds(sid * mseg + nzc * _ZB, zrem)], zsem).start()
    pltpu.sync_copy(idx_hbm.at[sid], idx_v)
    for k in range(8):
        for q in range(8):
            v = idx_v[k, pl.ds(q * _L, _L)]
            loc = v - base
            ok = (loc >= 0) & (loc < crange)
            idx_v[k, pl.ds(q * _L, _L)] = jnp.where(ok, loc, _DBIN)
    for j in range(nzc):
        pltpu.make_async_copy(
            z_v, m_s.at[pl.ds(sid * mseg + j * _ZB, _ZB)], zsem).wait()
    pltpu.make_async_copy(
        z_v.at[pl.ds(0, zrem)],
        m_s.at[pl.ds(sid * mseg + nzc * _ZB, zrem)], zsem).wait()
    plsc.subcore_barrier()

    for k in range(8):
        pltpu.sync_copy(ones_v, m_s.at[idx_v.at[k]], add=True)
    plsc.subcore_barrier()

    @pl.when(cid == 0)
    def _():
        seg = 499712 // _NS
        pltpu.sync_copy(m_s.at[pl.ds(sid * seg, seg)],
                        bounce_v.at[pl.ds(0, seg)])
        pltpu.sync_copy(bounce_v.at[pl.ds(0, seg)],
                        m_hbm.at[pl.ds(sid * seg, seg)])

    @pl.when(cid == 1)
    def _():
        seg = (_MPAD - _HALF) // _NS
        pltpu.sync_copy(m_s.at[pl.ds(sid * seg, seg)], bounce_v)
        pltpu.sync_copy(bounce_v,
                        m_hbm.at[pl.ds(_HALF + sid * seg, seg)])


_ASPLIT = 458752
_TCBLK = 2048
_NTCB = (999424 - _ASPLIT) // _TCBLK


@functools.partial(
    pl.kernel,
    mesh=_mesh,
    out_type=jax.ShapeDtypeStruct((_NW, _D, _L), jnp.float32),
    scratch_types=[
        pltpu.VMEM((2, _D, _W), jnp.float32),
        pltpu.VMEM((_W,), jnp.float32),
        pltpu.VMEM((_D, _L), jnp.float32),
        pltpu.SemaphoreType.DMA,
    ],
    compiler_params=pltpu.CompilerParams(use_tc_tiling_on_sc=True),
)
def _scan_sc(m_hbm, table_hbm, tail_hbm, out_hbm, tbuf, mbuf, part_v, sem):
    cid = lax.axis_index("c")
    sid = lax.axis_index("s")
    wid = sid * _NC + cid

    def pzero(i, _):
        part_v[i] = jnp.zeros((_L,), jnp.float32)
        return 0
    lax.fori_loop(0, _D, pzero, 0)

    def chunk_col(k):
        return (wid + k * _NW) * _W

    def fire(k):
        pltpu.make_async_copy(
            table_hbm.at[:, pl.ds(chunk_col(k), _W)], tbuf.at[jnp.mod(k, 2)],
            sem).start()

    def accumulate(p, col0):
        pltpu.sync_copy(m_hbm.at[pl.ds(col0, _W)], mbuf)
        mv = tuple(mbuf[pl.ds(j * _L, _L)] for j in range(_W // _L))

        def dgroup(dg, _):
            for u in range(4):
                d = dg * 4 + u
                acc = tbuf[p, d, pl.ds(0, _L)] * mv[0]
                for j in range(1, _W // _L):
                    acc = acc + tbuf[p, d, pl.ds(j * _L, _L)] * mv[j]
                plsc.addupdate(part_v.at[d], acc)
            return 0
        lax.fori_loop(0, _D // 4, dgroup, 0)

    kpw = _ASPLIT // (_W * _NW)
    fire(0)

    def scan_body(k, _):
        pltpu.make_async_copy(
            table_hbm.at[:, pl.ds(0, _W)], tbuf.at[jnp.mod(k, 2)], sem
        ).wait()

        @pl.when(k + 1 < kpw)
        def _():
            fire(k + 1)
        accumulate(jnp.mod(k, 2), chunk_col(k))
        return 0
    lax.fori_loop(0, kpw, scan_body, 0)

    @pl.when((cid == 1) & (sid == 0))
    def _():
        col0 = 999424
        pltpu.make_async_copy(
            table_hbm.at[:, pl.ds(col0, _W)], tbuf.at[0], sem).start()
        pltpu.make_async_copy(
            table_hbm.at[:, pl.ds(col0, _W)], tbuf.at[0], sem).wait()
        accumulate(0, col0)

    @pl.when((cid == 1) & (sid == 1))
    def _():
        col0 = 999936
        pltpu.make_async_copy(
            tail_hbm, tbuf.at[0].at[:, pl.ds(0, 128)], sem).start()
        pltpu.make_async_copy(
            tail_hbm, tbuf.at[0].at[:, pl.ds(0, 128)], sem).wait()
        pltpu.sync_copy(m_hbm.at[pl.ds(col0, 128)], mbuf.at[pl.ds(0, 128)])
        mv = tuple(mbuf[pl.ds(j * _L, _L)] for j in range(128 // _L))

        def dtail(dg, _):
            for u in range(4):
                d = dg * 4 + u
                acc = tbuf[0, d, pl.ds(0, _L)] * mv[0]
                for j in range(1, 128 // _L):
                    acc = acc + tbuf[0, d, pl.ds(j * _L, _L)] * mv[j]
                plsc.addupdate(part_v.at[d], acc)
            return 0
        lax.fori_loop(0, _D // 4, dtail, 0)

    pltpu.sync_copy(part_v, out_hbm.at[wid])


def _tc_matvec_body(t_ref, m_ref, o_ref):
    i = pl.program_id(0)

    @pl.when(i == 0)
    def _():
        o_ref[...] = jnp.zeros_like(o_ref)
    o_ref[...] += jnp.dot(t_ref[...], m_ref[...])[None, :]


def _tail_body(p_ref, t_ref, w_ref, b_ref, o_ref):
    tot = jnp.sum(p_ref[...], axis=(0, 2)) + t_ref[0, :]
    h = 1.0 / (1.0 + jnp.exp(-(tot * (1.0 / _B))))
    logits = jnp.sum(w_ref[...] * h[None, :], axis=1, keepdims=True)
    logits = logits + b_ref[...]
    s = 1.0 / (1.0 + jnp.exp(-logits))
    row = lax.broadcasted_iota(jnp.int32, (8, 1), 0)
    e = jnp.where(row < 2, jnp.exp(s), 0.0)
    o_ref[...] = e / jnp.sum(e)


def kernel(X, emb, W, b):
    idx = X.astype(jnp.int32).reshape(_NS, 8, 128)
    tail = jnp.zeros((_D, 128), jnp.float32).at[:, :_TAILC].set(
        emb[999936:].T)
    tbl = emb.T
    m = _build_m(idx)
    tc_part = pl.pallas_call(
        _tc_matvec_body,
        grid=(_NTCB,),
        in_specs=[
            pl.BlockSpec((_D, _TCBLK),
                         lambda i: (0, _ASPLIT // _TCBLK + i)),
            pl.BlockSpec((_TCBLK,), lambda i: (_ASPLIT // _TCBLK + i,)),
        ],
        out_specs=pl.BlockSpec((1, _D), lambda i: (0, 0)),
        out_shape=jax.ShapeDtypeStruct((1, _D), jnp.float32),
    )(tbl, m)
    partials = _scan_sc(m, tbl, tail)
    wp = jnp.zeros((8, _D), jnp.float32).at[:2].set(W)
    bp = jnp.zeros((8, 1), jnp.float32).at[:2, 0].set(b)
    out = pl.pallas_call(
        _tail_body,
        out_shape=jax.ShapeDtypeStruct((8, 1), jnp.float32),
    )(partials, tc_part, wp, bp)
    return out[:2, 0]


# speedup vs baseline: 2.0503x; 1.2850x over previous
import functools

import jax
import jax.numpy as jnp
from jax import lax
from jax.experimental import pallas as pl
from jax.experimental.pallas import tpu as pltpu
from jax.experimental.pallas import tpu_sc as plsc

_D = 64
_B = 16384
_NROW = 1_000_000
_MPAD = 1_000_064
_NC = 2
_NS = 16
_NW = _NC * _NS
_L = 16
_W = 512
_HALF = 499712
_TAILC = 64
_MLOC = 500608
_DBIN = 500480
_ZB = 8192

_mesh = plsc.VectorSubcoreMesh(core_axis_name="c", subcore_axis_name="s")


@functools.partial(
    pl.kernel,
    mesh=_mesh,
    out_type=jax.ShapeDtypeStruct((_MPAD,), jnp.float32),
    scratch_types=[
        pltpu.VMEM_SHARED((_MLOC,), jnp.float32),
        pltpu.VMEM((8, 128), jnp.int32),
        pltpu.VMEM((128,), jnp.float32),
        pltpu.VMEM((_ZB,), jnp.float32),
        pltpu.VMEM((31272,), jnp.float32),
        pltpu.SemaphoreType.DMA,
    ],
    compiler_params=pltpu.CompilerParams(use_tc_tiling_on_sc=True),
)
def _build_m(idx_hbm, m_hbm, m_s, idx_v, ones_v, z_v, bounce_v, zsem):
    cid = lax.axis_index("c")
    sid = lax.axis_index("s")
    wid = sid * _NC + cid
    base = cid * _HALF
    crange = jnp.where(cid == 0, _HALF, _NROW - _HALF)

    def zfill(i, _):
        z_v[pl.ds(i * _L, _L)] = jnp.zeros((_L,), jnp.float32)
        return 0
    lax.fori_loop(0, _ZB // _L, zfill, 0)
    for j in range(8):
        ones_v[pl.ds(j * _L, _L)] = jnp.ones((_L,), jnp.float32)

    mseg = _MLOC // _NS
    nzc = mseg // _ZB
    zrem = mseg - nzc * _ZB
    for j in range(nzc):
        pltpu.make_async_copy(
            z_v, m_s.at[pl.ds(sid * mseg + j * _ZB, _ZB)], zsem).start()
    pltpu.make_async_copy(
        z_v.at[pl.ds(0, zrem)],
        m_s.at[pl.ds(sid * mseg + nzc * _ZB, zrem)], zsem).start()
    pltpu.sync_copy(idx_hbm.at[sid], idx_v)
    for k in range(8):
        for q in range(8):
            v = idx_v[k, pl.ds(q * _L, _L)]
            loc = v - base
            ok = (loc >= 0) & (loc < crange)
            idx_v[k, pl.ds(q * _L, _L)] = jnp.where(ok, loc, _DBIN)
    for j in range(nzc):
        pltpu.make_async_copy(
            z_v, m_s.at[pl.ds(sid * mseg + j * _ZB, _ZB)], zsem).wait()
    pltpu.make_async_copy(
        z_v.at[pl.ds(0, zrem)],
        m_s.at[pl.ds(sid * mseg + nzc * _ZB, zrem)], zsem).wait()
    plsc.subcore_barrier()

    for k in range(8):
        pltpu.sync_copy(ones_v, m_s.at[idx_v.at[k]], add=True)
    plsc.subcore_barrier()

    @pl.when(cid == 0)
    def _():
        seg = 499712 // _NS
        pltpu.sync_copy(m_s.at[pl.ds(sid * seg, seg)],
                        bounce_v.at[pl.ds(0, seg)])
        pltpu.sync_copy(bounce_v.at[pl.ds(0, seg)],
                        m_hbm.at[pl.ds(sid * seg, seg)])

    @pl.when(cid == 1)
    def _():
        seg = (_MPAD - _HALF) // _NS
        pltpu.sync_copy(m_s.at[pl.ds(sid * seg, seg)], bounce_v)
        pltpu.sync_copy(bounce_v,
                        m_hbm.at[pl.ds(_HALF + sid * seg, seg)])


_ASPLIT = 344064
_TCBLK = 4096
_NTCB = (999424 - _ASPLIT) // _TCBLK


@functools.partial(
    pl.kernel,
    mesh=_mesh,
    out_type=jax.ShapeDtypeStruct((_NW, _D, _L), jnp.float32),
    scratch_types=[
        pltpu.VMEM((2, _D, _W), jnp.float32),
        pltpu.VMEM((_W,), jnp.float32),
        pltpu.VMEM((_D, _L), jnp.float32),
        pltpu.SemaphoreType.DMA,
    ],
    compiler_params=pltpu.CompilerParams(use_tc_tiling_on_sc=True),
)
def _scan_sc(m_hbm, table_hbm, tail_hbm, out_hbm, tbuf, mbuf, part_v, sem):
    cid = lax.axis_index("c")
    sid = lax.axis_index("s")
    wid = sid * _NC + cid

    def pzero(i, _):
        part_v[i] = jnp.zeros((_L,), jnp.float32)
        return 0
    lax.fori_loop(0, _D, pzero, 0)

    def chunk_col(k):
        return (wid + k * _NW) * _W

    def fire(k):
        pltpu.make_async_copy(
            table_hbm.at[:, pl.ds(chunk_col(k), _W)], tbuf.at[jnp.mod(k, 2)],
            sem).start()

    def accumulate(p, col0):
        pltpu.sync_copy(m_hbm.at[pl.ds(col0, _W)], mbuf)
        mv = tuple(mbuf[pl.ds(j * _L, _L)] for j in range(_W // _L))

        def dgroup(dg, _):
            for u in range(4):
                d = dg * 4 + u
                acc = tbuf[p, d, pl.ds(0, _L)] * mv[0]
                for j in range(1, _W // _L):
                    acc = acc + tbuf[p, d, pl.ds(j * _L, _L)] * mv[j]
                plsc.addupdate(part_v.at[d], acc)
            return 0
        lax.fori_loop(0, _D // 4, dgroup, 0)

    kpw = _ASPLIT // (_W * _NW)
    fire(0)

    def scan_body(k, _):
        pltpu.make_async_copy(
            table_hbm.at[:, pl.ds(0, _W)], tbuf.at[jnp.mod(k, 2)], sem
        ).wait()

        @pl.when(k + 1 < kpw)
        def _():
            fire(k + 1)
        accumulate(jnp.mod(k, 2), chunk_col(k))
        return 0
    lax.fori_loop(0, kpw, scan_body, 0)

    @pl.when((cid == 1) & (sid == 0))
    def _():
        col0 = 999424
        pltpu.make_async_copy(
            table_hbm.at[:, pl.ds(col0, _W)], tbuf.at[0], sem).start()
        pltpu.make_async_copy(
            table_hbm.at[:, pl.ds(col0, _W)], tbuf.at[0], sem).wait()
        accumulate(0, col0)

    @pl.when((cid == 1) & (sid == 1))
    def _():
        col0 = 999936
        pltpu.make_async_copy(
            tail_hbm, tbuf.at[0].at[:, pl.ds(0, 128)], sem).start()
        pltpu.make_async_copy(
            tail_hbm, tbuf.at[0].at[:, pl.ds(0, 128)], sem).wait()
        pltpu.sync_copy(m_hbm.at[pl.ds(col0, 128)], mbuf.at[pl.ds(0, 128)])
        mv = tuple(mbuf[pl.ds(j * _L, _L)] for j in range(128 // _L))

        def dtail(dg, _):
            for u in range(4):
                d = dg * 4 + u
                acc = tbuf[0, d, pl.ds(0, _L)] * mv[0]
                for j in range(1, 128 // _L):
                    acc = acc + tbuf[0, d, pl.ds(j * _L, _L)] * mv[j]
                plsc.addupdate(part_v.at[d], acc)
            return 0
        lax.fori_loop(0, _D // 4, dtail, 0)

    pltpu.sync_copy(part_v, out_hbm.at[wid])


def _tc_matvec_body(t_ref, m_ref, o_ref):
    i = pl.program_id(0)

    @pl.when(i == 0)
    def _():
        o_ref[...] = jnp.zeros_like(o_ref)
    o_ref[...] += jnp.sum(t_ref[...] * m_ref[...][None, :], axis=1)[None, :]


def _tail_body(p_ref, t_ref, w_ref, b_ref, o_ref):
    tot = jnp.sum(p_ref[...], axis=(0, 2)) + t_ref[0, :]
    h = 1.0 / (1.0 + jnp.exp(-(tot * (1.0 / _B))))
    logits = jnp.sum(w_ref[...] * h[None, :], axis=1, keepdims=True)
    logits = logits + b_ref[...]
    s = 1.0 / (1.0 + jnp.exp(-logits))
    row = lax.broadcasted_iota(jnp.int32, (8, 1), 0)
    e = jnp.where(row < 2, jnp.exp(s), 0.0)
    o_ref[...] = e / jnp.sum(e)


def kernel(X, emb, W, b):
    idx = X.astype(jnp.int32).reshape(_NS, 8, 128)
    tail = jnp.zeros((_D, 128), jnp.float32).at[:, :_TAILC].set(
        emb[999936:].T)
    tbl = emb.T
    m = _build_m(idx)
    tc_part = pl.pallas_call(
        _tc_matvec_body,
        grid=(_NTCB,),
        in_specs=[
            pl.BlockSpec((_D, _TCBLK),
                         lambda i: (0, _ASPLIT // _TCBLK + i)),
            pl.BlockSpec((_TCBLK,), lambda i: (_ASPLIT // _TCBLK + i,)),
        ],
        out_specs=pl.BlockSpec((1, _D), lambda i: (0, 0)),
        out_shape=jax.ShapeDtypeStruct((1, _D), jnp.float32),
    )(tbl, m)
    partials = _scan_sc(m, tbl, tail)
    wp = jnp.zeros((8, _D), jnp.float32).at[:2].set(W)
    bp = jnp.zeros((8, 1), jnp.float32).at[:2, 0].set(b)
    out = pl.pallas_call(
        _tail_body,
        out_shape=jax.ShapeDtypeStruct((8, 1), jnp.float32),
    )(partials, tc_part, wp, bp)
    return out[:2, 0]


# speedup vs baseline: 2.1554x; 1.0513x over previous
import functools

import jax
import jax.numpy as jnp
from jax import lax
from jax.experimental import pallas as pl
from jax.experimental.pallas import tpu as pltpu
from jax.experimental.pallas import tpu_sc as plsc

_D = 64
_B = 16384
_NROW = 1_000_000
_MPAD = 1_000_064
_NC = 2
_NS = 16
_NW = _NC * _NS
_L = 16
_W = 512
_HALF = 499712
_TAILC = 64
_MLOC = 500608
_DBIN = 500480
_ZB = 8192

_mesh = plsc.VectorSubcoreMesh(core_axis_name="c", subcore_axis_name="s")


@functools.partial(
    pl.kernel,
    mesh=_mesh,
    out_type=jax.ShapeDtypeStruct((_MPAD,), jnp.float32),
    scratch_types=[
        pltpu.VMEM_SHARED((_MLOC,), jnp.float32),
        pltpu.VMEM((8, 128), jnp.int32),
        pltpu.VMEM((128,), jnp.float32),
        pltpu.VMEM((_ZB,), jnp.float32),
        pltpu.VMEM((31272,), jnp.float32),
        pltpu.SemaphoreType.DMA,
    ],
    compiler_params=pltpu.CompilerParams(use_tc_tiling_on_sc=True),
)
def _build_m(idx_hbm, m_hbm, m_s, idx_v, ones_v, z_v, bounce_v, zsem):
    cid = lax.axis_index("c")
    sid = lax.axis_index("s")
    wid = sid * _NC + cid
    base = cid * _HALF
    crange = jnp.where(cid == 0, _HALF, _NROW - _HALF)

    def zfill(i, _):
        z_v[pl.ds(i * _L, _L)] = jnp.zeros((_L,), jnp.float32)
        return 0
    lax.fori_loop(0, _ZB // _L, zfill, 0)
    for j in range(8):
        ones_v[pl.ds(j * _L, _L)] = jnp.ones((_L,), jnp.float32)

    mseg = _MLOC // _NS
    nzc = mseg // _ZB
    zrem = mseg - nzc * _ZB
    for j in range(nzc):
        pltpu.make_async_copy(
            z_v, m_s.at[pl.ds(sid * mseg + j * _ZB, _ZB)], zsem).start()
    pltpu.make_async_copy(
        z_v.at[pl.ds(0, zrem)],
        m_s.at[pl.ds(sid * mseg + nzc * _ZB, zrem)], zsem).start()
    pltpu.sync_copy(idx_hbm.at[sid], idx_v)
    for k in range(8):
        for q in range(8):
            v = idx_v[k, pl.ds(q * _L, _L)]
            loc = v - base
            ok = (loc >= 0) & (loc < crange)
            idx_v[k, pl.ds(q * _L, _L)] = jnp.where(ok, loc, _DBIN)
    for j in range(nzc):
        pltpu.make_async_copy(
            z_v, m_s.at[pl.ds(sid * mseg + j * _ZB, _ZB)], zsem).wait()
    pltpu.make_async_copy(
        z_v.at[pl.ds(0, zrem)],
        m_s.at[pl.ds(sid * mseg + nzc * _ZB, zrem)], zsem).wait()
    plsc.subcore_barrier()

    for k in range(8):
        pltpu.sync_copy(ones_v, m_s.at[idx_v.at[k]], add=True)
    plsc.subcore_barrier()

    @pl.when(cid == 0)
    def _():
        seg = 499712 // _NS
        pltpu.sync_copy(m_s.at[pl.ds(sid * seg, seg)],
                        bounce_v.at[pl.ds(0, seg)])
        pltpu.sync_copy(bounce_v.at[pl.ds(0, seg)],
                        m_hbm.at[pl.ds(sid * seg, seg)])

    @pl.when(cid == 1)
    def _():
        seg = (_MPAD - _HALF) // _NS
        pltpu.sync_copy(m_s.at[pl.ds(sid * seg, seg)], bounce_v)
        pltpu.sync_copy(bounce_v,
                        m_hbm.at[pl.ds(_HALF + sid * seg, seg)])


_ASPLIT = 344064
_TCBLK = 4096
_NTCB = (999424 - _ASPLIT) // _TCBLK


@functools.partial(
    pl.kernel,
    mesh=_mesh,
    out_type=jax.ShapeDtypeStruct((_NW, _D, _L), jnp.float32),
    scratch_types=[
        pltpu.VMEM((2, _D, _W), jnp.float32),
        pltpu.VMEM((_W,), jnp.float32),
        pltpu.VMEM((_D, _L), jnp.float32),
        pltpu.SemaphoreType.DMA,
    ],
    compiler_params=pltpu.CompilerParams(use_tc_tiling_on_sc=True),
)
def _scan_sc(m_hbm, table_hbm, tail_hbm, out_hbm, tbuf, mbuf, part_v, sem):
    cid = lax.axis_index("c")
    sid = lax.axis_index("s")
    wid = sid * _NC + cid

    def pzero(i, _):
        part_v[i] = jnp.zeros((_L,), jnp.float32)
        return 0
    lax.fori_loop(0, _D, pzero, 0)

    def chunk_col(k):
        return (wid + k * _NW) * _W

    def fire(k):
        pltpu.make_async_copy(
            table_hbm.at[:, pl.ds(chunk_col(k), _W)], tbuf.at[jnp.mod(k, 2)],
            sem).start()

    def accumulate(p, col0):
        pltpu.sync_copy(m_hbm.at[pl.ds(col0, _W)], mbuf)
        mv = tuple(mbuf[pl.ds(j * _L, _L)] for j in range(_W // _L))

        def dgroup(dg, _):
            for u in range(4):
                d = dg * 4 + u
                acc = tbuf[p, d, pl.ds(0, _L)] * mv[0]
                for j in range(1, _W // _L):
                    acc = acc + tbuf[p, d, pl.ds(j * _L, _L)] * mv[j]
                plsc.addupdate(part_v.at[d], acc)
            return 0
        lax.fori_loop(0, _D // 4, dgroup, 0)

    kpw = _ASPLIT // (_W * _NW)
    fire(0)

    def scan_body(k, _):
        pltpu.make_async_copy(
            table_hbm.at[:, pl.ds(0, _W)], tbuf.at[jnp.mod(k, 2)], sem
        ).wait()

        @pl.when(k + 1 < kpw)
        def _():
            fire(k + 1)
        accumulate(jnp.mod(k, 2), chunk_col(k))
        return 0
    lax.fori_loop(0, kpw, scan_body, 0)

    @pl.when((cid == 1) & (sid == 0))
    def _():
        col0 = 999424
        pltpu.make_async_copy(
            table_hbm.at[:, pl.ds(col0, _W)], tbuf.at[0], sem).start()
        pltpu.make_async_copy(
            table_hbm.at[:, pl.ds(col0, _W)], tbuf.at[0], sem).wait()
        accumulate(0, col0)

    @pl.when((cid == 1) & (sid == 1))
    def _():
        col0 = 999936
        pltpu.make_async_copy(
            tail_hbm, tbuf.at[0].at[:, pl.ds(0, 128)], sem).start()
        pltpu.make_async_copy(
            tail_hbm, tbuf.at[0].at[:, pl.ds(0, 128)], sem).wait()
        pltpu.sync_copy(m_hbm.at[pl.ds(col0, 128)], mbuf.at[pl.ds(0, 128)])
        mv = tuple(mbuf[pl.ds(j * _L, _L)] for j in range(128 // _L))

        def dtail(dg, _):
            for u in range(4):
                d = dg * 4 + u
                acc = tbuf[0, d, pl.ds(0, _L)] * mv[0]
                for j in range(1, 128 // _L):
                    acc = acc + tbuf[0, d, pl.ds(j * _L, _L)] * mv[j]
                plsc.addupdate(part_v.at[d], acc)
            return 0
        lax.fori_loop(0, _D // 4, dtail, 0)

    pltpu.sync_copy(part_v, out_hbm.at[wid])


def _tc_matvec_body(t_ref, m_ref, o_ref):
    i = pl.program_id(0)

    @pl.when(i == 0)
    def _():
        o_ref[...] = jnp.zeros_like(o_ref)
    t = t_ref[...]
    m = m_ref[...]
    acc = o_ref[...]
    for g in range(_TCBLK // 128):
        acc = acc + t[:, g * 128:(g + 1) * 128] * m[g * 128:(g + 1) * 128][None, :]
    o_ref[...] = acc


def _tail_body(p_ref, t_ref, w_ref, b_ref, o_ref):
    tot = jnp.sum(p_ref[...], axis=(0, 2)) + jnp.sum(t_ref[...], axis=1)
    h = 1.0 / (1.0 + jnp.exp(-(tot * (1.0 / _B))))
    logits = jnp.sum(w_ref[...] * h[None, :], axis=1, keepdims=True)
    logits = logits + b_ref[...]
    s = 1.0 / (1.0 + jnp.exp(-logits))
    row = lax.broadcasted_iota(jnp.int32, (8, 1), 0)
    e = jnp.where(row < 2, jnp.exp(s), 0.0)
    o_ref[...] = e / jnp.sum(e)


def kernel(X, emb, W, b):
    idx = X.astype(jnp.int32).reshape(_NS, 8, 128)
    tail = jnp.zeros((_D, 128), jnp.float32).at[:, :_TAILC].set(
        emb[999936:].T)
    tbl = emb.T
    m = _build_m(idx)
    tc_part = pl.pallas_call(
        _tc_matvec_body,
        grid=(_NTCB,),
        in_specs=[
            pl.BlockSpec((_D, _TCBLK),
                         lambda i: (0, _ASPLIT // _TCBLK + i)),
            pl.BlockSpec((_TCBLK,), lambda i: (_ASPLIT // _TCBLK + i,)),
        ],
        out_specs=pl.BlockSpec((_D, 128), lambda i: (0, 0)),
        out_shape=jax.ShapeDtypeStruct((_D, 128), jnp.float32),
    )(tbl, m)
    partials = _scan_sc(m, tbl, tail)
    wp = jnp.zeros((8, _D), jnp.float32).at[:2].set(W)
    bp = jnp.zeros((8, 1), jnp.float32).at[:2, 0].set(b)
    out = pl.pallas_call(
        _tail_body,
        out_shape=jax.ShapeDtypeStruct((8, 1), jnp.float32),
    )(partials, tc_part, wp, bp)
    return out[:2, 0]


# speedup vs baseline: 2.4743x; 1.1479x over previous
import functools

import jax
import jax.numpy as jnp
from jax import lax
from jax.experimental import pallas as pl
from jax.experimental.pallas import tpu as pltpu
from jax.experimental.pallas import tpu_sc as plsc

_D = 64
_B = 16384
_NROW = 1_000_000
_MPAD = 1_000_064
_NC = 2
_NS = 16
_NW = _NC * _NS
_L = 16
_W = 512
_HALF = 499712
_TAILC = 64
_MLOC = 500608
_DBIN = 500480
_ZB = 8192

_mesh = plsc.VectorSubcoreMesh(core_axis_name="c", subcore_axis_name="s")


@functools.partial(
    pl.kernel,
    mesh=_mesh,
    out_type=jax.ShapeDtypeStruct((_MPAD,), jnp.float32),
    scratch_types=[
        pltpu.VMEM_SHARED((_MLOC,), jnp.float32),
        pltpu.VMEM((8, 128), jnp.int32),
        pltpu.VMEM((128,), jnp.float32),
        pltpu.VMEM((_ZB,), jnp.float32),
        pltpu.VMEM((31272,), jnp.float32),
        pltpu.SemaphoreType.DMA,
    ],
    compiler_params=pltpu.CompilerParams(use_tc_tiling_on_sc=True),
)
def _build_m(idx_hbm, m_hbm, m_s, idx_v, ones_v, z_v, bounce_v, zsem):
    cid = lax.axis_index("c")
    sid = lax.axis_index("s")
    wid = sid * _NC + cid
    base = cid * _HALF
    crange = jnp.where(cid == 0, _HALF, _NROW - _HALF)

    def zfill(i, _):
        z_v[pl.ds(i * _L, _L)] = jnp.zeros((_L,), jnp.float32)
        return 0
    lax.fori_loop(0, _ZB // _L, zfill, 0)
    for j in range(8):
        ones_v[pl.ds(j * _L, _L)] = jnp.ones((_L,), jnp.float32)

    mseg = _MLOC // _NS
    nzc = mseg // _ZB
    zrem = mseg - nzc * _ZB
    for j in range(nzc):
        pltpu.make_async_copy(
            z_v, m_s.at[pl.ds(sid * mseg + j * _ZB, _ZB)], zsem).start()
    pltpu.make_async_copy(
        z_v.at[pl.ds(0, zrem)],
        m_s.at[pl.ds(sid * mseg + nzc * _ZB, zrem)], zsem).start()
    pltpu.sync_copy(idx_hbm.at[sid], idx_v)
    for k in range(8):
        for q in range(8):
            v = idx_v[k, pl.ds(q * _L, _L)]
            loc = v - base
            ok = (loc >= 0) & (loc < crange)
            idx_v[k, pl.ds(q * _L, _L)] = jnp.where(ok, loc, _DBIN)
    for j in range(nzc):
        pltpu.make_async_copy(
            z_v, m_s.at[pl.ds(sid * mseg + j * _ZB, _ZB)], zsem).wait()
    pltpu.make_async_copy(
        z_v.at[pl.ds(0, zrem)],
        m_s.at[pl.ds(sid * mseg + nzc * _ZB, zrem)], zsem).wait()
    plsc.subcore_barrier()

    for k in range(8):
        pltpu.sync_copy(ones_v, m_s.at[idx_v.at[k]], add=True)
    plsc.subcore_barrier()

    @pl.when(cid == 0)
    def _():
        seg = 499712 // _NS
        pltpu.sync_copy(m_s.at[pl.ds(sid * seg, seg)],
                        bounce_v.at[pl.ds(0, seg)])
        pltpu.sync_copy(bounce_v.at[pl.ds(0, seg)],
                        m_hbm.at[pl.ds(sid * seg, seg)])

    @pl.when(cid == 1)
    def _():
        seg = (_MPAD - _HALF) // _NS
        pltpu.sync_copy(m_s.at[pl.ds(sid * seg, seg)], bounce_v)
        pltpu.sync_copy(bounce_v,
                        m_hbm.at[pl.ds(_HALF + sid * seg, seg)])


_ASPLIT = 442368
_TCBLK = 8192
_NTCB = (999424 - _ASPLIT) // _TCBLK


@functools.partial(
    pl.kernel,
    mesh=_mesh,
    out_type=jax.ShapeDtypeStruct((_NW, _D, _L), jnp.float32),
    scratch_types=[
        pltpu.VMEM((2, _D, _W), jnp.float32),
        pltpu.VMEM((_W,), jnp.float32),
        pltpu.VMEM((_D, _L), jnp.float32),
        pltpu.SemaphoreType.DMA,
    ],
    compiler_params=pltpu.CompilerParams(use_tc_tiling_on_sc=True),
)
def _scan_sc(m_hbm, table_hbm, tail_hbm, out_hbm, tbuf, mbuf, part_v, sem):
    cid = lax.axis_index("c")
    sid = lax.axis_index("s")
    wid = sid * _NC + cid

    def pzero(i, _):
        part_v[i] = jnp.zeros((_L,), jnp.float32)
        return 0
    lax.fori_loop(0, _D, pzero, 0)

    def chunk_col(k):
        return (wid + k * _NW) * _W

    def fire(k):
        pltpu.make_async_copy(
            table_hbm.at[:, pl.ds(chunk_col(k), _W)], tbuf.at[jnp.mod(k, 2)],
            sem).start()

    def accumulate(p, col0):
        pltpu.sync_copy(m_hbm.at[pl.ds(col0, _W)], mbuf)
        mv = tuple(mbuf[pl.ds(j * _L, _L)] for j in range(_W // _L))

        def dgroup(dg, _):
            for u in range(4):
                d = dg * 4 + u
                acc = tbuf[p, d, pl.ds(0, _L)] * mv[0]
                for j in range(1, _W // _L):
                    acc = acc + tbuf[p, d, pl.ds(j * _L, _L)] * mv[j]
                plsc.addupdate(part_v.at[d], acc)
            return 0
        lax.fori_loop(0, _D // 4, dgroup, 0)

    kpw = _ASPLIT // (_W * _NW)
    fire(0)

    def scan_body(k, _):
        pltpu.make_async_copy(
            table_hbm.at[:, pl.ds(0, _W)], tbuf.at[jnp.mod(k, 2)], sem
        ).wait()

        @pl.when(k + 1 < kpw)
        def _():
            fire(k + 1)
        accumulate(jnp.mod(k, 2), chunk_col(k))
        return 0
    lax.fori_loop(0, kpw, scan_body, 0)

    @pl.when((cid == 1) & (sid == 0))
    def _():
        col0 = 999424
        pltpu.make_async_copy(
            table_hbm.at[:, pl.ds(col0, _W)], tbuf.at[0], sem).start()
        pltpu.make_async_copy(
            table_hbm.at[:, pl.ds(col0, _W)], tbuf.at[0], sem).wait()
        accumulate(0, col0)

    @pl.when((cid == 1) & (sid == 1))
    def _():
        col0 = 999936
        pltpu.make_async_copy(
            tail_hbm, tbuf.at[0].at[:, pl.ds(0, 128)], sem).start()
        pltpu.make_async_copy(
            tail_hbm, tbuf.at[0].at[:, pl.ds(0, 128)], sem).wait()
        pltpu.sync_copy(m_hbm.at[pl.ds(col0, 128)], mbuf.at[pl.ds(0, 128)])
        mv = tuple(mbuf[pl.ds(j * _L, _L)] for j in range(128 // _L))

        def dtail(dg, _):
            for u in range(4):
                d = dg * 4 + u
                acc = tbuf[0, d, pl.ds(0, _L)] * mv[0]
                for j in range(1, 128 // _L):
                    acc = acc + tbuf[0, d, pl.ds(j * _L, _L)] * mv[j]
                plsc.addupdate(part_v.at[d], acc)
            return 0
        lax.fori_loop(0, _D // 4, dtail, 0)

    pltpu.sync_copy(part_v, out_hbm.at[wid])


def _tc_matvec_body(t_ref, m_ref, o_ref):
    i = pl.program_id(0)

    @pl.when(i == 0)
    def _():
        o_ref[...] = jnp.zeros_like(o_ref)
    t = t_ref[...]
    m = m_ref[...]
    ng = _TCBLK // 128
    acc0 = o_ref[...]
    acc1 = jnp.zeros_like(acc0)
    for g in range(0, ng, 2):
        acc0 = acc0 + t[:, g * 128:(g + 1) * 128] * m[g * 128:(g + 1) * 128][None, :]
        acc1 = acc1 + t[:, (g + 1) * 128:(g + 2) * 128] * m[(g + 1) * 128:(g + 2) * 128][None, :]
    o_ref[...] = acc0 + acc1


def _tail_body(p_ref, t_ref, w_ref, b_ref, o_ref):
    tot = jnp.sum(p_ref[...], axis=(0, 2)) + jnp.sum(t_ref[...], axis=1)
    h = 1.0 / (1.0 + jnp.exp(-(tot * (1.0 / _B))))
    logits = jnp.sum(w_ref[...] * h[None, :], axis=1, keepdims=True)
    logits = logits + b_ref[...]
    s = 1.0 / (1.0 + jnp.exp(-logits))
    row = lax.broadcasted_iota(jnp.int32, (8, 1), 0)
    e = jnp.where(row < 2, jnp.exp(s), 0.0)
    o_ref[...] = e / jnp.sum(e)


def kernel(X, emb, W, b):
    idx = X.astype(jnp.int32).reshape(_NS, 8, 128)
    tail = jnp.zeros((_D, 128), jnp.float32).at[:, :_TAILC].set(
        emb[999936:].T)
    tbl = emb.T
    m = _build_m(idx)
    tc_part = pl.pallas_call(
        _tc_matvec_body,
        grid=(_NTCB,),
        in_specs=[
            pl.BlockSpec((_D, _TCBLK),
                         lambda i: (0, _ASPLIT // _TCBLK + i)),
            pl.BlockSpec((_TCBLK,), lambda i: (_ASPLIT // _TCBLK + i,)),
        ],
        out_specs=pl.BlockSpec((_D, 128), lambda i: (0, 0)),
        out_shape=jax.ShapeDtypeStruct((_D, 128), jnp.float32),
        compiler_params=pltpu.CompilerParams(
            dimension_semantics=("arbitrary",)),
    )(tbl, m)
    partials = _scan_sc(m, tbl, tail)
    wp = jnp.zeros((8, _D), jnp.float32).at[:2].set(W)
    bp = jnp.zeros((8, 1), jnp.float32).at[:2, 0].set(b)
    out = pl.pallas_call(
        _tail_body,
        out_shape=jax.ShapeDtypeStruct((8, 1), jnp.float32),
    )(partials, tc_part, wp, bp)
    return out[:2, 0]


# speedup vs baseline: 2.7129x; 1.0964x over previous
import functools

import jax
import jax.numpy as jnp
from jax import lax
from jax.experimental import pallas as pl
from jax.experimental.pallas import tpu as pltpu
from jax.experimental.pallas import tpu_sc as plsc

_D = 64
_B = 16384
_NROW = 1_000_000
_MPAD = 1_000_064
_NC = 2
_NS = 16
_NW = _NC * _NS
_L = 16
_W = 512
_HALF = 499712
_TAILC = 64
_MLOC = 500608
_DBIN = 500480
_ZB = 8192

_mesh = plsc.VectorSubcoreMesh(core_axis_name="c", subcore_axis_name="s")


@functools.partial(
    pl.kernel,
    mesh=_mesh,
    out_type=jax.ShapeDtypeStruct((_MPAD,), jnp.float32),
    scratch_types=[
        pltpu.VMEM_SHARED((_MLOC,), jnp.float32),
        pltpu.VMEM((8, 128), jnp.int32),
        pltpu.VMEM((128,), jnp.float32),
        pltpu.VMEM((_ZB,), jnp.float32),
        pltpu.VMEM((31272,), jnp.float32),
        pltpu.SemaphoreType.DMA,
    ],
    compiler_params=pltpu.CompilerParams(use_tc_tiling_on_sc=True),
)
def _build_m(idx_hbm, m_hbm, m_s, idx_v, ones_v, z_v, bounce_v, zsem):
    cid = lax.axis_index("c")
    sid = lax.axis_index("s")
    wid = sid * _NC + cid
    base = cid * _HALF
    crange = jnp.where(cid == 0, _HALF, _NROW - _HALF)

    def zfill(i, _):
        z_v[pl.ds(i * _L, _L)] = jnp.zeros((_L,), jnp.float32)
        return 0
    lax.fori_loop(0, _ZB // _L, zfill, 0)
    for j in range(8):
        ones_v[pl.ds(j * _L, _L)] = jnp.ones((_L,), jnp.float32)

    mseg = _MLOC // _NS
    nzc = mseg // _ZB
    zrem = mseg - nzc * _ZB
    for j in range(nzc):
        pltpu.make_async_copy(
            z_v, m_s.at[pl.ds(sid * mseg + j * _ZB, _ZB)], zsem).start()
    pltpu.make_async_copy(
        z_v.at[pl.ds(0, zrem)],
        m_s.at[pl.ds(sid * mseg + nzc * _ZB, zrem)], zsem).start()
    pltpu.sync_copy(idx_hbm.at[sid], idx_v)
    for k in range(8):
        for q in range(8):
            v = idx_v[k, pl.ds(q * _L, _L)]
            loc = v - base
            ok = (loc >= 0) & (loc < crange)
            idx_v[k, pl.ds(q * _L, _L)] = jnp.where(ok, loc, _DBIN)
    for j in range(nzc):
        pltpu.make_async_copy(
            z_v, m_s.at[pl.ds(sid * mseg + j * _ZB, _ZB)], zsem).wait()
    pltpu.make_async_copy(
        z_v.at[pl.ds(0, zrem)],
        m_s.at[pl.ds(sid * mseg + nzc * _ZB, zrem)], zsem).wait()
    plsc.subcore_barrier()

    for k in range(8):
        pltpu.sync_copy(ones_v, m_s.at[idx_v.at[k]], add=True)
    plsc.subcore_barrier()

    @pl.when(cid == 0)
    def _():
        seg = 499712 // _NS
        pltpu.sync_copy(m_s.at[pl.ds(sid * seg, seg)],
                        bounce_v.at[pl.ds(0, seg)])
        pltpu.sync_copy(bounce_v.at[pl.ds(0, seg)],
                        m_hbm.at[pl.ds(sid * seg, seg)])

    @pl.when(cid == 1)
    def _():
        seg = (_MPAD - _HALF) // _NS
        pltpu.sync_copy(m_s.at[pl.ds(sid * seg, seg)], bounce_v)
        pltpu.sync_copy(bounce_v,
                        m_hbm.at[pl.ds(_HALF + sid * seg, seg)])


_ASPLIT = 376832
_TCBLK = 8192
_NTCB = (999424 - _ASPLIT) // _TCBLK


@functools.partial(
    pl.kernel,
    mesh=_mesh,
    out_type=jax.ShapeDtypeStruct((_NW, _D, _L), jnp.float32),
    scratch_types=[
        pltpu.VMEM((2, _D, _W), jnp.float32),
        pltpu.VMEM((_W,), jnp.float32),
        pltpu.VMEM((_D, _L), jnp.float32),
        pltpu.SemaphoreType.DMA,
    ],
    compiler_params=pltpu.CompilerParams(use_tc_tiling_on_sc=True),
)
def _scan_sc(m_hbm, table_hbm, tail_hbm, out_hbm, tbuf, mbuf, part_v, sem):
    cid = lax.axis_index("c")
    sid = lax.axis_index("s")
    wid = sid * _NC + cid

    def pzero(i, _):
        part_v[i] = jnp.zeros((_L,), jnp.float32)
        return 0
    lax.fori_loop(0, _D, pzero, 0)

    def chunk_col(k):
        return (wid + k * _NW) * _W

    def fire(k):
        pltpu.make_async_copy(
            table_hbm.at[:, pl.ds(chunk_col(k), _W)], tbuf.at[jnp.mod(k, 2)],
            sem).start()

    def accumulate(p, col0):
        pltpu.sync_copy(m_hbm.at[pl.ds(col0, _W)], mbuf)
        mv = tuple(mbuf[pl.ds(j * _L, _L)] for j in range(_W // _L))

        def dgroup(dg, _):
            for u in range(4):
                d = dg * 4 + u
                acc = tbuf[p, d, pl.ds(0, _L)] * mv[0]
                for j in range(1, _W // _L):
                    acc = acc + tbuf[p, d, pl.ds(j * _L, _L)] * mv[j]
                plsc.addupdate(part_v.at[d], acc)
            return 0
        lax.fori_loop(0, _D // 4, dgroup, 0)

    kpw = _ASPLIT // (_W * _NW)
    fire(0)

    def scan_body(k, _):
        pltpu.make_async_copy(
            table_hbm.at[:, pl.ds(0, _W)], tbuf.at[jnp.mod(k, 2)], sem
        ).wait()

        @pl.when(k + 1 < kpw)
        def _():
            fire(k + 1)
        accumulate(jnp.mod(k, 2), chunk_col(k))
        return 0
    lax.fori_loop(0, kpw, scan_body, 0)

    @pl.when((cid == 1) & (sid == 0))
    def _():
        col0 = 999424
        pltpu.make_async_copy(
            table_hbm.at[:, pl.ds(col0, _W)], tbuf.at[0], sem).start()
        pltpu.make_async_copy(
            table_hbm.at[:, pl.ds(col0, _W)], tbuf.at[0], sem).wait()
        accumulate(0, col0)

    @pl.when((cid == 1) & (sid == 1))
    def _():
        col0 = 999936
        pltpu.make_async_copy(
            tail_hbm, tbuf.at[0].at[:, pl.ds(0, 128)], sem).start()
        pltpu.make_async_copy(
            tail_hbm, tbuf.at[0].at[:, pl.ds(0, 128)], sem).wait()
        pltpu.sync_copy(m_hbm.at[pl.ds(col0, 128)], mbuf.at[pl.ds(0, 128)])
        mv = tuple(mbuf[pl.ds(j * _L, _L)] for j in range(128 // _L))

        def dtail(dg, _):
            for u in range(4):
                d = dg * 4 + u
                acc = tbuf[0, d, pl.ds(0, _L)] * mv[0]
                for j in range(1, 128 // _L):
                    acc = acc + tbuf[0, d, pl.ds(j * _L, _L)] * mv[j]
                plsc.addupdate(part_v.at[d], acc)
            return 0
        lax.fori_loop(0, _D // 4, dtail, 0)

    pltpu.sync_copy(part_v, out_hbm.at[wid])


def _tc_matvec_body(t_ref, m_ref, o_ref):
    i = pl.program_id(0)

    @pl.when(i == 0)
    def _():
        o_ref[...] = jnp.zeros_like(o_ref)
    t = t_ref[...]
    m = m_ref[...]
    ng = _TCBLK // 128
    acc0 = o_ref[...]
    acc1 = jnp.zeros_like(acc0)
    for g in range(0, ng, 2):
        acc0 = acc0 + t[:, g * 128:(g + 1) * 128] * m[g * 128:(g + 1) * 128][None, :]
        acc1 = acc1 + t[:, (g + 1) * 128:(g + 2) * 128] * m[(g + 1) * 128:(g + 2) * 128][None, :]
    o_ref[...] = acc0 + acc1


def _tail_body(p_ref, t_ref, w_ref, b_ref, o_ref):
    tot = jnp.sum(p_ref[...], axis=(0, 2)) + jnp.sum(t_ref[...], axis=1)
    h = 1.0 / (1.0 + jnp.exp(-(tot * (1.0 / _B))))
    logits = jnp.sum(w_ref[...] * h[None, :], axis=1, keepdims=True)
    logits = logits + b_ref[...]
    s = 1.0 / (1.0 + jnp.exp(-logits))
    row = lax.broadcasted_iota(jnp.int32, (8, 1), 0)
    e = jnp.where(row < 2, jnp.exp(s), 0.0)
    o_ref[...] = e / jnp.sum(e)


def kernel(X, emb, W, b):
    idx = X.astype(jnp.int32).reshape(_NS, 8, 128)
    tail = jnp.zeros((_D, 128), jnp.float32).at[:, :_TAILC].set(
        emb[999936:].T)
    tbl = emb.T
    m = _build_m(idx)
    tc_part = pl.pallas_call(
        _tc_matvec_body,
        grid=(_NTCB,),
        in_specs=[
            pl.BlockSpec((_D, _TCBLK),
                         lambda i: (0, _ASPLIT // _TCBLK + i)),
            pl.BlockSpec((_TCBLK,), lambda i: (_ASPLIT // _TCBLK + i,)),
        ],
        out_specs=pl.BlockSpec((_D, 128), lambda i: (0, 0)),
        out_shape=jax.ShapeDtypeStruct((_D, 128), jnp.float32),
        compiler_params=pltpu.CompilerParams(
            dimension_semantics=("arbitrary",)),
    )(tbl, m)
    partials = _scan_sc(m, tbl, tail)
    wp = jnp.zeros((8, _D), jnp.float32).at[:2].set(W)
    bp = jnp.zeros((8, 1), jnp.float32).at[:2, 0].set(b)
    out = pl.pallas_call(
        _tail_body,
        out_shape=jax.ShapeDtypeStruct((8, 1), jnp.float32),
    )(partials, tc_part, wp, bp)
    return out[:2, 0]
